# baseline (speedup 1.0000x reference)
import functools

import jax
import jax.numpy as jnp
from jax import lax
from jax.experimental import pallas as pl
from jax.experimental.pallas import tpu as pltpu
from jax.experimental.pallas import tpu_sc as plsc

N = 10000
E = 320000
D = 128
NC = 2
NS = 16
CH = 128
KC = 8
NCHUNK = 160
E_PAD = NS * NCHUNK * CH
N_PAD = 10112
ROWS_PER_TILE = N_PAD // NS

_mesh = plsc.VectorSubcoreMesh(core_axis_name="c", subcore_axis_name="s")


NSB = NCHUNK // KC


def _sc_agg_body(x_hbm, src_hbm, dst_hbm, zeros_hbm,
                 agg_out,
                 src_a, dst_a, src_b, dst_b, rows0, rows1,
                 acc_sh, sem00, sem01, sem10, sem11, sem_i):
    c = lax.axis_index("c")
    s = lax.axis_index("s")
    r0 = s * ROWS_PER_TILE
    rows = (rows0, rows1)
    sems = ((sem00, sem01), (sem10, sem11))
    pltpu.sync_copy(zeros_hbm.at[pl.ds(r0, ROWS_PER_TILE)],
                    acc_sh.at[pl.ds(r0, ROWS_PER_TILE)])
    plsc.subcore_barrier()

    def start_rows(src_buf, j, b):
        pltpu.async_copy(x_hbm.at[src_buf.at[j, pl.ds(0, CH // 2)]],
                         rows[b].at[pl.ds(0, CH // 2)], sems[b][0])
        pltpu.async_copy(x_hbm.at[src_buf.at[j, pl.ds(CH // 2, CH // 2)]],
                         rows[b].at[pl.ds(CH // 2, CH // 2)], sems[b][1])

    def wait_rows(b):
        pltpu.make_async_copy(x_hbm.at[pl.ds(0, CH // 2)],
                              rows[b].at[pl.ds(0, CH // 2)], sems[b][0]).wait()
        pltpu.make_async_copy(x_hbm.at[pl.ds(0, CH // 2)],
                              rows[b].at[pl.ds(CH // 2, CH // 2)], sems[b][1]).wait()

    def wait_idx(src_n, dst_n):
        pltpu.make_async_copy(src_hbm.at[c, s, pl.ds(0, KC)], src_n, sem_i).wait()
        pltpu.make_async_copy(dst_hbm.at[c, s, pl.ds(0, KC)], dst_n, sem_i).wait()

    def superblock(g, src_c, dst_c, src_n, dst_n):
        @pl.when(g + 1 < NSB)
        def _():
            pltpu.async_copy(src_hbm.at[c, s, pl.ds((g + 1) * KC, KC)], src_n, sem_i)
            pltpu.async_copy(dst_hbm.at[c, s, pl.ds((g + 1) * KC, KC)], dst_n, sem_i)
        for j in range(KC):
            b = j & 1
            nb = (j + 1) & 1
            if j + 1 < KC:
                start_rows(src_c, j + 1, nb)
            else:
                @pl.when(g + 1 < NSB)
                def _():
                    wait_idx(src_n, dst_n)
                    start_rows(src_n, 0, nb)
            wait_rows(b)
            pltpu.sync_copy(rows[b], acc_sh.at[dst_c.at[j]], add=True)

    pltpu.sync_copy(src_hbm.at[c, s, pl.ds(0, KC)], src_a)
    pltpu.sync_copy(dst_hbm.at[c, s, pl.ds(0, KC)], dst_a)
    start_rows(src_a, 0, 0)

    def outer(t, carry):
        superblock(2 * t, src_a, dst_a, src_b, dst_b)
        superblock(2 * t + 1, src_b, dst_b, src_a, dst_a)
        return carry

    lax.fori_loop(0, NSB // 2, outer, 0)
    plsc.subcore_barrier()
    pltpu.sync_copy(acc_sh.at[pl.ds(r0, ROWS_PER_TILE)],
                    agg_out.at[c, pl.ds(r0, ROWS_PER_TILE)])


_sc_agg = pl.kernel(
    _sc_agg_body,
    out_type=jax.ShapeDtypeStruct((NC, N_PAD, D), jnp.float32),
    mesh=_mesh,
    scratch_types=[
        pltpu.VMEM((KC, CH), jnp.int32),
        pltpu.VMEM((KC, CH), jnp.int32),
        pltpu.VMEM((KC, CH), jnp.int32),
        pltpu.VMEM((KC, CH), jnp.int32),
        pltpu.VMEM((CH, D), jnp.float32),
        pltpu.VMEM((CH, D), jnp.float32),
        pltpu.VMEM_SHARED((N_PAD, D), jnp.float32),
        pltpu.SemaphoreType.DMA,
        pltpu.SemaphoreType.DMA,
        pltpu.SemaphoreType.DMA,
        pltpu.SemaphoreType.DMA,
        pltpu.SemaphoreType.DMA,
    ],
)


def _sc_deg_body(ones_hbm, dst_hbm, zeros_hbm,
                 deg_out,
                 dst_v, ones_v, acc_sh):
    c = lax.axis_index("c")
    s = lax.axis_index("s")
    r0 = s * ROWS_PER_TILE
    pltpu.sync_copy(zeros_hbm.at[pl.ds(r0, ROWS_PER_TILE)],
                    acc_sh.at[pl.ds(r0, ROWS_PER_TILE)])
    pltpu.sync_copy(ones_hbm, ones_v)
    plsc.subcore_barrier()

    def outer(g, carry):
        pltpu.sync_copy(dst_hbm.at[c, s, pl.ds(g * KC, KC)], dst_v)

        def body(j, carry2):
            pltpu.sync_copy(ones_v, acc_sh.at[dst_v.at[j]], add=True)
            return carry2

        lax.fori_loop(0, KC, body, 0)
        return carry

    lax.fori_loop(0, NCHUNK // KC, outer, 0)
    plsc.subcore_barrier()
    pltpu.sync_copy(acc_sh.at[pl.ds(r0, ROWS_PER_TILE)],
                    deg_out.at[c, pl.ds(r0, ROWS_PER_TILE)])


_sc_deg = pl.kernel(
    _sc_deg_body,
    out_type=jax.ShapeDtypeStruct((NC, N_PAD, D), jnp.float32),
    mesh=_mesh,
    scratch_types=[
        pltpu.VMEM((KC, CH), jnp.int32),
        pltpu.VMEM((CH, D), jnp.float32),
        pltpu.VMEM_SHARED((N_PAD, D), jnp.float32),
    ],
)


def _dense_body(relu, x_ref, agg_ref, deg_ref, ws_ref, wn_ref, b_ref, out_ref):
    deg = jnp.maximum(deg_ref[0], 1.0)
    a = agg_ref[0] / deg
    h = (jnp.dot(x_ref[...], ws_ref[0], preferred_element_type=jnp.float32)
         + jnp.dot(a, wn_ref[0], preferred_element_type=jnp.float32)
         + b_ref[0])
    out_ref[...] = jnp.maximum(h, 0.0) if relu else h


def _dense(x, agg, deg, ws, wn, b, relu):
    B = 1000
    grid = (2, N // B)
    nb = N // B
    return pl.pallas_call(
        functools.partial(_dense_body, relu),
        grid=grid,
        in_specs=[
            pl.BlockSpec((B, D), lambda t, i: (t * nb + i, 0)),
            pl.BlockSpec((1, B, D), lambda t, i: (t, i, 0)),
            pl.BlockSpec((1, B, 1), lambda t, i: (t, i, 0)),
            pl.BlockSpec((1, D, D), lambda t, i: (t, 0, 0)),
            pl.BlockSpec((1, D, D), lambda t, i: (t, 0, 0)),
            pl.BlockSpec((1, 1, D), lambda t, i: (t, 0, 0)),
        ],
        out_specs=pl.BlockSpec((B, D), lambda t, i: (t * nb + i, 0)),
        out_shape=jax.ShapeDtypeStruct((2 * N, D), jnp.float32),
    )(x, agg, deg, ws, wn, b)


def _prep_rel(ei, src_off):
    src = ei[0].astype(jnp.int32) + src_off
    dst = ei[1].astype(jnp.int32)
    pad = E_PAD - E
    src = jnp.concatenate([src, jnp.zeros((pad,), jnp.int32)])
    dst = jnp.concatenate([dst, jnp.full((pad,), N, jnp.int32)])
    return src.reshape(NS, NCHUNK, CH), dst.reshape(NS, NCHUNK, CH)


def kernel(x_user, x_movie, edge_index_rates, edge_index_rated_by,
           W_self_rates1, W_neigh_rates1, b_rates1,
           W_self_rb1, W_neigh_rb1, b_rb1,
           W_self_rates2, W_neigh_rates2, b_rates2,
           W_self_rb2, W_neigh_rb2, b_rb2):
    src_rb, dst_rb = _prep_rel(edge_index_rated_by, N)
    src_rt, dst_rt = _prep_rel(edge_index_rates, 0)
    src_idx = jnp.stack([src_rb, src_rt])
    dst_idx = jnp.stack([dst_rb, dst_rt])
    zeros = jnp.zeros((N_PAD, D), jnp.float32)
    ones = jnp.ones((CH, D), jnp.float32)

    deg = _sc_deg(ones, dst_idx, zeros)[:, :, :1]

    x_all = jnp.concatenate([x_user, x_movie], axis=0)
    agg1 = _sc_agg(x_all, src_idx, dst_idx, zeros)

    ws1 = jnp.stack([W_self_rb1, W_self_rates1])
    wn1 = jnp.stack([W_neigh_rb1, W_neigh_rates1])
    bs1 = jnp.stack([b_rb1, b_rates1])[:, None, :]
    h = _dense(x_all, agg1, deg, ws1, wn1, bs1, relu=True)

    agg2 = _sc_agg(h, src_idx, dst_idx, zeros)

    ws2 = jnp.stack([W_self_rb2, W_self_rates2])
    wn2 = jnp.stack([W_neigh_rb2, W_neigh_rates2])
    bs2 = jnp.stack([b_rb2, b_rates2])[:, None, :]
    out = _dense(h, agg2, deg, ws2, wn2, bs2, relu=False)
    return out[:N], out[N:]


# speedup vs baseline: 1.0033x; 1.0033x over previous
import functools

import jax
import jax.numpy as jnp
from jax import lax
from jax.experimental import pallas as pl
from jax.experimental.pallas import tpu as pltpu
from jax.experimental.pallas import tpu_sc as plsc

N = 10000
E = 320000
D = 128
NC = 2
NS = 16
CH = 128
KC = 8
NCHUNK = 160
E_PAD = NS * NCHUNK * CH
N_PAD = 10112
ROWS_PER_TILE = N_PAD // NS

_mesh = plsc.VectorSubcoreMesh(core_axis_name="c", subcore_axis_name="s")


NSB = NCHUNK // KC


def _sc_agg_body(x_hbm, src_hbm, dst_hbm, zeros_hbm,
                 agg_out,
                 src_a, dst_a, src_b, dst_b, rows0, rows1,
                 acc_sh, sem00, sem01, sem10, sem11, sem_i):
    c = lax.axis_index("c")
    s = lax.axis_index("s")
    r0 = s * ROWS_PER_TILE
    rows = (rows0, rows1)
    sems = ((sem00, sem01), (sem10, sem11))
    pltpu.sync_copy(zeros_hbm.at[pl.ds(r0, ROWS_PER_TILE)],
                    acc_sh.at[pl.ds(r0, ROWS_PER_TILE)])
    plsc.subcore_barrier()

    def start_rows(src_buf, j, b):
        pltpu.async_copy(x_hbm.at[src_buf.at[j]], rows[b], sems[b][0])

    def wait_rows(b):
        pltpu.make_async_copy(x_hbm.at[pl.ds(0, CH)], rows[b], sems[b][0]).wait()

    def wait_idx(src_n, dst_n):
        pltpu.make_async_copy(src_hbm.at[c, s, pl.ds(0, KC)], src_n, sem_i).wait()
        pltpu.make_async_copy(dst_hbm.at[c, s, pl.ds(0, KC)], dst_n, sem_i).wait()

    def superblock(g, src_c, dst_c, src_n, dst_n):
        @pl.when(g + 1 < NSB)
        def _():
            pltpu.async_copy(src_hbm.at[c, s, pl.ds((g + 1) * KC, KC)], src_n, sem_i)
            pltpu.async_copy(dst_hbm.at[c, s, pl.ds((g + 1) * KC, KC)], dst_n, sem_i)
        for j in range(KC):
            b = j & 1
            nb = (j + 1) & 1
            if j + 1 < KC:
                start_rows(src_c, j + 1, nb)
            else:
                @pl.when(g + 1 < NSB)
                def _():
                    wait_idx(src_n, dst_n)
                    start_rows(src_n, 0, nb)
            wait_rows(b)
            pltpu.sync_copy(rows[b], acc_sh.at[dst_c.at[j]], add=True)

    pltpu.sync_copy(src_hbm.at[c, s, pl.ds(0, KC)], src_a)
    pltpu.sync_copy(dst_hbm.at[c, s, pl.ds(0, KC)], dst_a)
    start_rows(src_a, 0, 0)

    def outer(t, carry):
        superblock(2 * t, src_a, dst_a, src_b, dst_b)
        superblock(2 * t + 1, src_b, dst_b, src_a, dst_a)
        return carry

    lax.fori_loop(0, NSB // 2, outer, 0)
    plsc.subcore_barrier()
    pltpu.sync_copy(acc_sh.at[pl.ds(r0, ROWS_PER_TILE)],
                    agg_out.at[c, pl.ds(r0, ROWS_PER_TILE)])


_sc_agg = pl.kernel(
    _sc_agg_body,
    out_type=jax.ShapeDtypeStruct((NC, N_PAD, D), jnp.float32),
    mesh=_mesh,
    scratch_types=[
        pltpu.VMEM((KC, CH), jnp.int32),
        pltpu.VMEM((KC, CH), jnp.int32),
        pltpu.VMEM((KC, CH), jnp.int32),
        pltpu.VMEM((KC, CH), jnp.int32),
        pltpu.VMEM((CH, D), jnp.float32),
        pltpu.VMEM((CH, D), jnp.float32),
        pltpu.VMEM_SHARED((N_PAD, D), jnp.float32),
        pltpu.SemaphoreType.DMA,
        pltpu.SemaphoreType.DMA,
        pltpu.SemaphoreType.DMA,
        pltpu.SemaphoreType.DMA,
        pltpu.SemaphoreType.DMA,
    ],
)


def _sc_deg_body(ones_hbm, dst_hbm, zeros_hbm,
                 deg_out,
                 dst_v, ones_v, acc_sh):
    c = lax.axis_index("c")
    s = lax.axis_index("s")
    r0 = s * ROWS_PER_TILE
    pltpu.sync_copy(zeros_hbm.at[pl.ds(r0, ROWS_PER_TILE)],
                    acc_sh.at[pl.ds(r0, ROWS_PER_TILE)])
    pltpu.sync_copy(ones_hbm, ones_v)
    plsc.subcore_barrier()

    def outer(g, carry):
        pltpu.sync_copy(dst_hbm.at[c, s, pl.ds(g * KC, KC)], dst_v)

        def body(j, carry2):
            pltpu.sync_copy(ones_v, acc_sh.at[dst_v.at[j]], add=True)
            return carry2

        lax.fori_loop(0, KC, body, 0)
        return carry

    lax.fori_loop(0, NCHUNK // KC, outer, 0)
    plsc.subcore_barrier()
    pltpu.sync_copy(acc_sh.at[pl.ds(r0, ROWS_PER_TILE)],
                    deg_out.at[c, pl.ds(r0, ROWS_PER_TILE)])


_sc_deg = pl.kernel(
    _sc_deg_body,
    out_type=jax.ShapeDtypeStruct((NC, N_PAD, D), jnp.float32),
    mesh=_mesh,
    scratch_types=[
        pltpu.VMEM((KC, CH), jnp.int32),
        pltpu.VMEM((CH, D), jnp.float32),
        pltpu.VMEM_SHARED((N_PAD, D), jnp.float32),
    ],
)


def _dense_body(relu, x_ref, agg_ref, deg_ref, ws_ref, wn_ref, b_ref, out_ref):
    deg = jnp.maximum(deg_ref[0], 1.0)
    a = agg_ref[0] / deg
    h = (jnp.dot(x_ref[...], ws_ref[0], preferred_element_type=jnp.float32)
         + jnp.dot(a, wn_ref[0], preferred_element_type=jnp.float32)
         + b_ref[0])
    out_ref[...] = jnp.maximum(h, 0.0) if relu else h


def _dense(x, agg, deg, ws, wn, b, relu):
    B = 1000
    grid = (2, N // B)
    nb = N // B
    return pl.pallas_call(
        functools.partial(_dense_body, relu),
        grid=grid,
        in_specs=[
            pl.BlockSpec((B, D), lambda t, i: (t * nb + i, 0)),
            pl.BlockSpec((1, B, D), lambda t, i: (t, i, 0)),
            pl.BlockSpec((1, B, 1), lambda t, i: (t, i, 0)),
            pl.BlockSpec((1, D, D), lambda t, i: (t, 0, 0)),
            pl.BlockSpec((1, D, D), lambda t, i: (t, 0, 0)),
            pl.BlockSpec((1, 1, D), lambda t, i: (t, 0, 0)),
        ],
        out_specs=pl.BlockSpec((B, D), lambda t, i: (t * nb + i, 0)),
        out_shape=jax.ShapeDtypeStruct((2 * N, D), jnp.float32),
    )(x, agg, deg, ws, wn, b)


def _prep_rel(ei, src_off):
    src = ei[0].astype(jnp.int32) + src_off
    dst = ei[1].astype(jnp.int32)
    pad = E_PAD - E
    src = jnp.concatenate([src, jnp.zeros((pad,), jnp.int32)])
    dst = jnp.concatenate([dst, jnp.full((pad,), N, jnp.int32)])
    return src.reshape(NS, NCHUNK, CH), dst.reshape(NS, NCHUNK, CH)


def kernel(x_user, x_movie, edge_index_rates, edge_index_rated_by,
           W_self_rates1, W_neigh_rates1, b_rates1,
           W_self_rb1, W_neigh_rb1, b_rb1,
           W_self_rates2, W_neigh_rates2, b_rates2,
           W_self_rb2, W_neigh_rb2, b_rb2):
    src_rb, dst_rb = _prep_rel(edge_index_rated_by, N)
    src_rt, dst_rt = _prep_rel(edge_index_rates, 0)
    src_idx = jnp.stack([src_rb, src_rt])
    dst_idx = jnp.stack([dst_rb, dst_rt])
    zeros = jnp.zeros((N_PAD, D), jnp.float32)
    ones = jnp.ones((CH, D), jnp.float32)

    deg = _sc_deg(ones, dst_idx, zeros)[:, :, :1]

    x_all = jnp.concatenate([x_user, x_movie], axis=0)
    agg1 = _sc_agg(x_all, src_idx, dst_idx, zeros)

    ws1 = jnp.stack([W_self_rb1, W_self_rates1])
    wn1 = jnp.stack([W_neigh_rb1, W_neigh_rates1])
    bs1 = jnp.stack([b_rb1, b_rates1])[:, None, :]
    h = _dense(x_all, agg1, deg, ws1, wn1, bs1, relu=True)

    agg2 = _sc_agg(h, src_idx, dst_idx, zeros)

    ws2 = jnp.stack([W_self_rb2, W_self_rates2])
    wn2 = jnp.stack([W_neigh_rb2, W_neigh_rates2])
    bs2 = jnp.stack([b_rb2, b_rates2])[:, None, :]
    out = _dense(h, agg2, deg, ws2, wn2, bs2, relu=False)
    return out[:N], out[N:]


# speedup vs baseline: 1.0039x; 1.0006x over previous
import functools

import jax
import jax.numpy as jnp
from jax import lax
from jax.experimental import pallas as pl
from jax.experimental.pallas import tpu as pltpu
from jax.experimental.pallas import tpu_sc as plsc

N = 10000
E = 320000
D = 128
NC = 2
NS = 16
CH = 128
KC = 8
NCHUNK = 160
E_PAD = NS * NCHUNK * CH
N_PAD = 10112
ROWS_PER_TILE = N_PAD // NS

_mesh = plsc.VectorSubcoreMesh(core_axis_name="c", subcore_axis_name="s")


NSB = NCHUNK // KC


def _sc_agg_body(x_hbm, src_hbm, dst_hbm, zeros_hbm,
                 agg_out,
                 src_a, dst_a, src_b, dst_b, rows0, rows1,
                 acc_sh, sem0, sem1, sem_i):
    c = lax.axis_index("c")
    s = lax.axis_index("s")
    r0 = s * ROWS_PER_TILE
    rows = (rows0, rows1)
    sems = (sem0, sem1)
    pltpu.sync_copy(zeros_hbm.at[pl.ds(r0, ROWS_PER_TILE)],
                    acc_sh.at[pl.ds(r0, ROWS_PER_TILE)])
    plsc.subcore_barrier()

    def start_rows(src_buf, j, b):
        pltpu.async_copy(x_hbm.at[src_buf.at[j]], rows[b], sems[b])

    def wait_rows(b):
        pltpu.make_async_copy(x_hbm.at[pl.ds(0, CH)], rows[b], sems[b]).wait()

    def wait_idx(src_n, dst_n):
        pltpu.make_async_copy(src_hbm.at[c, s, pl.ds(0, KC)], src_n, sem_i).wait()
        pltpu.make_async_copy(dst_hbm.at[c, s, pl.ds(0, KC)], dst_n, sem_i).wait()

    def superblock(g, src_c, dst_c, src_n, dst_n):
        @pl.when(g + 1 < NSB)
        def _():
            pltpu.async_copy(src_hbm.at[c, s, pl.ds((g + 1) * KC, KC)], src_n, sem_i)
            pltpu.async_copy(dst_hbm.at[c, s, pl.ds((g + 1) * KC, KC)], dst_n, sem_i)
        for j in range(KC):
            b = j & 1
            nb = (j + 1) & 1
            if j + 1 < KC:
                start_rows(src_c, j + 1, nb)
            else:
                @pl.when(g + 1 < NSB)
                def _():
                    wait_idx(src_n, dst_n)
                    start_rows(src_n, 0, nb)
            wait_rows(b)
            pltpu.sync_copy(rows[b], acc_sh.at[dst_c.at[j]], add=True)

    pltpu.sync_copy(src_hbm.at[c, s, pl.ds(0, KC)], src_a)
    pltpu.sync_copy(dst_hbm.at[c, s, pl.ds(0, KC)], dst_a)
    start_rows(src_a, 0, 0)

    def outer(t, carry):
        superblock(2 * t, src_a, dst_a, src_b, dst_b)
        superblock(2 * t + 1, src_b, dst_b, src_a, dst_a)
        return carry

    lax.fori_loop(0, NSB // 2, outer, 0)
    plsc.subcore_barrier()
    pltpu.sync_copy(acc_sh.at[pl.ds(r0, ROWS_PER_TILE)],
                    agg_out.at[c, pl.ds(r0, ROWS_PER_TILE)])


_sc_agg = pl.kernel(
    _sc_agg_body,
    out_type=jax.ShapeDtypeStruct((NC, N_PAD, D), jnp.float32),
    mesh=_mesh,
    scratch_types=[
        pltpu.VMEM((KC, CH), jnp.int32),
        pltpu.VMEM((KC, CH), jnp.int32),
        pltpu.VMEM((KC, CH), jnp.int32),
        pltpu.VMEM((KC, CH), jnp.int32),
        pltpu.VMEM((CH, D), jnp.float32),
        pltpu.VMEM((CH, D), jnp.float32),
        pltpu.VMEM_SHARED((N_PAD, D), jnp.float32),
        pltpu.SemaphoreType.DMA,
        pltpu.SemaphoreType.DMA,
        pltpu.SemaphoreType.DMA,
    ],
)


def _sc_deg_body(ones_hbm, dst_hbm, zeros_hbm,
                 deg_out,
                 dst_v, ones_v, acc_sh):
    c = lax.axis_index("c")
    s = lax.axis_index("s")
    r0 = s * ROWS_PER_TILE
    pltpu.sync_copy(zeros_hbm.at[pl.ds(r0, ROWS_PER_TILE)],
                    acc_sh.at[pl.ds(r0, ROWS_PER_TILE)])
    pltpu.sync_copy(ones_hbm, ones_v)
    plsc.subcore_barrier()

    def outer(g, carry):
        pltpu.sync_copy(dst_hbm.at[c, s, pl.ds(g * KC, KC)], dst_v)

        def body(j, carry2):
            pltpu.sync_copy(ones_v, acc_sh.at[dst_v.at[j]], add=True)
            return carry2

        lax.fori_loop(0, KC, body, 0)
        return carry

    lax.fori_loop(0, NCHUNK // KC, outer, 0)
    plsc.subcore_barrier()
    pltpu.sync_copy(acc_sh.at[pl.ds(r0, ROWS_PER_TILE)],
                    deg_out.at[c, pl.ds(r0, ROWS_PER_TILE)])


_sc_deg = pl.kernel(
    _sc_deg_body,
    out_type=jax.ShapeDtypeStruct((NC, N_PAD, D), jnp.float32),
    mesh=_mesh,
    scratch_types=[
        pltpu.VMEM((KC, CH), jnp.int32),
        pltpu.VMEM((CH, D), jnp.float32),
        pltpu.VMEM_SHARED((N_PAD, D), jnp.float32),
    ],
)


def _dense_body(relu, x_ref, agg_ref, deg_ref, ws_ref, wn_ref, b_ref, out_ref):
    deg = jnp.maximum(deg_ref[0], 1.0)
    a = agg_ref[0] / deg
    h = (jnp.dot(x_ref[...], ws_ref[0], preferred_element_type=jnp.float32)
         + jnp.dot(a, wn_ref[0], preferred_element_type=jnp.float32)
         + b_ref[0])
    out_ref[...] = jnp.maximum(h, 0.0) if relu else h


def _dense(x, agg, deg, ws, wn, b, relu):
    B = 1000
    grid = (2, N // B)
    nb = N // B
    return pl.pallas_call(
        functools.partial(_dense_body, relu),
        grid=grid,
        in_specs=[
            pl.BlockSpec((B, D), lambda t, i: (t * nb + i, 0)),
            pl.BlockSpec((1, B, D), lambda t, i: (t, i, 0)),
            pl.BlockSpec((1, B, 1), lambda t, i: (t, i, 0)),
            pl.BlockSpec((1, D, D), lambda t, i: (t, 0, 0)),
            pl.BlockSpec((1, D, D), lambda t, i: (t, 0, 0)),
            pl.BlockSpec((1, 1, D), lambda t, i: (t, 0, 0)),
        ],
        out_specs=pl.BlockSpec((B, D), lambda t, i: (t * nb + i, 0)),
        out_shape=jax.ShapeDtypeStruct((2 * N, D), jnp.float32),
    )(x, agg, deg, ws, wn, b)


def _prep_rel(ei, src_off):
    src = ei[0].astype(jnp.int32) + src_off
    dst = ei[1].astype(jnp.int32)
    pad = E_PAD - E
    src = jnp.concatenate([src, jnp.zeros((pad,), jnp.int32)])
    dst = jnp.concatenate([dst, jnp.full((pad,), N, jnp.int32)])
    return src.reshape(NS, NCHUNK, CH), dst.reshape(NS, NCHUNK, CH)


def kernel(x_user, x_movie, edge_index_rates, edge_index_rated_by,
           W_self_rates1, W_neigh_rates1, b_rates1,
           W_self_rb1, W_neigh_rb1, b_rb1,
           W_self_rates2, W_neigh_rates2, b_rates2,
           W_self_rb2, W_neigh_rb2, b_rb2):
    src_rb, dst_rb = _prep_rel(edge_index_rated_by, N)
    src_rt, dst_rt = _prep_rel(edge_index_rates, 0)
    src_idx = jnp.stack([src_rb, src_rt])
    dst_idx = jnp.stack([dst_rb, dst_rt])
    zeros = jnp.zeros((N_PAD, D), jnp.float32)
    ones = jnp.ones((CH, D), jnp.float32)

    deg = _sc_deg(ones, dst_idx, zeros)[:, :, :1]

    x_all = jnp.concatenate([x_user, x_movie], axis=0)
    agg1 = _sc_agg(x_all, src_idx, dst_idx, zeros)

    ws1 = jnp.stack([W_self_rb1, W_self_rates1])
    wn1 = jnp.stack([W_neigh_rb1, W_neigh_rates1])
    bs1 = jnp.stack([b_rb1, b_rates1])[:, None, :]
    h = _dense(x_all, agg1, deg, ws1, wn1, bs1, relu=True)

    agg2 = _sc_agg(h, src_idx, dst_idx, zeros)

    ws2 = jnp.stack([W_self_rb2, W_self_rates2])
    wn2 = jnp.stack([W_neigh_rb2, W_neigh_rates2])
    bs2 = jnp.stack([b_rb2, b_rates2])[:, None, :]
    out = _dense(h, agg2, deg, ws2, wn2, bs2, relu=False)
    return out[:N], out[N:]
